# baseline (speedup 1.0000x reference)
import functools
import math

import jax
import jax.numpy as jnp
from jax import lax
from jax.experimental import pallas as pl
from jax.experimental.pallas import tpu as pltpu
from jax.experimental.pallas import tpu_sc as plsc

_TAN30 = math.tan(math.pi / 6)
_SCALE = 1.0 / 0.56 - 1.0

N0, N1 = 5000, 10000
E0, E1 = 320000, 640000
H, LDIM, D, OUT = 64, 32, 3, 4
NP0, NP1 = 5120, 10016
EP0, EP1 = 327680, 655360
R0 = EP0 // 128 // 32
R1 = EP1 // 128 // 32
CCH = 2
CP0 = 5120
NQP = 10240
KNN_BQ = 512
GQ = 32768

_NS = 16


def _elu(x):
    return jnp.where(x > 0, x, jnp.exp(jnp.minimum(x, 0.0)) - 1.0)


def _bn(z, g, b):
    m = jnp.mean(z, axis=0, keepdims=True)
    zc = z - m
    v = jnp.mean(zc * zc, axis=0, keepdims=True)
    return zc * lax.rsqrt(v + 1e-5) * g + b


def _dot(a, w):
    return jnp.dot(a, w, preferred_element_type=jnp.float32)


def _tail(x, pos, wrx, wrp, wtx, wtp, br, y_o, r_o):
    y_o[...] = _dot(x, wrx) + _dot(pos, wrp)
    r_o[...] = _dot(x, wtx) + _dot(pos, wtp) + br


def _stage_a_body(lat, lW, lb, g, b, pos, wrx, wrp, wtx, wtp, br, y_o, r_o):
    z = _dot(lat[...], lW[...]) + lb[...]
    x = _elu(_bn(z, g[...], b[...]))
    _tail(x, pos[...], wrx[...], wrp[...], wtx[...], wtp[...], br[...], y_o, r_o)


def _stage_mid_body(agg, r, g, b, pos, wrx, wrp, wtx, wtp, br, y_o, r_o):
    z = agg[0] + agg[1] + r[...]
    x = _elu(_bn(z, g[...], b[...]))
    _tail(x, pos[...], wrx[...], wrp[...], wtx[...], wtp[...], br[...], y_o, r_o)


def _stage_x_body(agg, r, g, b, x_o):
    z = agg[0] + agg[1] + r[...]
    x_o[...] = _elu(_bn(z, g[...], b[...]))


def _stage_interp_body(g0, g1, g2, w, pos, wrx, wrp, wtx, wtp, br,
                       y_o, r_o):
    w0 = w[:, 0:1]
    w1 = w[:, 1:2]
    w2 = w[:, 2:3]
    x = (g0[...] * w0 + g1[...] * w1 + g2[...] * w2) / (w0 + w1 + w2)
    _tail(x, pos[...], wrx[...], wrp[...], wtx[...], wtp[...], br[...], y_o, r_o)


def _stage_fin_body(agg, r, o):
    o[...] = (agg[0] + agg[1] + r[...])[:, :OUT]


def _transform_rows(px, py, pz):
    nx = px - _TAN30 * py
    s = 1.0 + _SCALE * (py / 1.1963)
    return nx * s, py * s, pz * s


def _knn_body(q_ref, pt_ref, idx_o, w_o):
    q = q_ref[...]
    qx = q[:, 0:1] - _TAN30 * q[:, 1:2]
    qs = 1.0 + _SCALE * (q[:, 1:2] / 1.1963)
    tqx, tqy, tqz = qx * qs, q[:, 1:2] * qs, q[:, 2:3] * qs
    qn = tqx * tqx + tqy * tqy + tqz * tqz

    tx, ty, tz = _transform_rows(pt_ref[0:1, :], pt_ref[1:2, :], pt_ref[2:3, :])
    cn = tx * tx + ty * ty + tz * tz
    colr = lax.broadcasted_iota(jnp.int32, (1, CP0), 1)
    cn = jnp.where(colr < N0, cn, jnp.inf)
    tp = jnp.concatenate([tx, ty, tz], axis=0)
    tq = jnp.concatenate([tqx, tqy, tqz], axis=1)
    d = qn + cn - 2.0 * _dot(tq, tp)

    col = lax.broadcasted_iota(jnp.int32, (KNN_BQ, CP0), 1)
    cols = []
    for _ in range(3):
        m = jnp.min(d, axis=1, keepdims=True)
        i = jnp.min(jnp.where(d == m, col, jnp.int32(2**30)), axis=1,
                    keepdims=True)
        cols.append((i, 1.0 / jnp.clip(m, 1e-16, None)))
        d = jnp.where(col == i, jnp.inf, d)
    z = jnp.zeros_like(cols[0][0])
    idx_o[...] = jnp.concatenate(
        [cols[0][0], cols[1][0], cols[2][0], z, z, z, z, z], axis=1)
    zf = jnp.zeros_like(cols[0][1])
    w_o[...] = jnp.concatenate(
        [cols[0][1], cols[1][1], cols[2][1], zf, zf, zf, zf, zf], axis=1)


def _sds(shape, dtype=jnp.float32):
    return jax.ShapeDtypeStruct(shape, dtype)


def _call(body, out_shape, **kw):
    kw.setdefault("compiler_params",
                  pltpu.CompilerParams(vmem_limit_bytes=100 * 1024 * 1024))
    return pl.pallas_call(body, out_shape=out_shape, **kw)


def _make_segsum(NP, W, R):
    mesh = plsc.VectorSubcoreMesh(core_axis_name="c", subcore_axis_name="s")
    sl = NP // _NS

    @functools.partial(
        pl.kernel, mesh=mesh,
        out_type=_sds((2, NP, W)),
        compiler_params=pltpu.CompilerParams(use_tc_tiling_on_sc=False),
        scratch_types=[
            pltpu.VMEM((2, 2, 128), jnp.int32),
            pltpu.VMEM((2, 128, W), jnp.float32),
            pltpu.VMEM_SHARED((NP, W), jnp.float32),
            pltpu.SemaphoreType.DMA,
        ],
    )
    def k(edges, y, zeros, out, idx_v, rows_v, acc, sem_g):
        c = lax.axis_index("c")
        s = lax.axis_index("s")
        wid = c * _NS + s
        base = wid * R
        pltpu.sync_copy(zeros.at[pl.ds(s * sl, sl)], acc.at[pl.ds(s * sl, sl)])
        plsc.subcore_barrier()

        def body(g, carry):
            pltpu.sync_copy(edges.at[base + g], idx_v.at[0])
            pltpu.async_copy(y.at[idx_v.at[0, 0]], rows_v.at[0], sem_g).wait()
            pltpu.sync_copy(rows_v.at[0], acc.at[idx_v.at[0, 1]], add=True)
            return carry

        lax.fori_loop(0, R, body, 0)
        plsc.subcore_barrier()
        pltpu.sync_copy(acc.at[pl.ds(s * sl, sl)], out.at[c, pl.ds(s * sl, sl)])

    return k


def _make_knn_gather():
    mesh = plsc.VectorSubcoreMesh(core_axis_name="c", subcore_axis_name="s")
    RG = GQ // 128 // 32

    @functools.partial(
        pl.kernel, mesh=mesh,
        out_type=_sds((GQ, H)),
        compiler_params=pltpu.CompilerParams(use_tc_tiling_on_sc=False),
        scratch_types=[
            pltpu.VMEM((RG, 128), jnp.int32),
            pltpu.VMEM((RG, 128, H), jnp.float32),
            pltpu.SemaphoreType.DMA,
            pltpu.SemaphoreType.DMA,
        ],
    )
    def k(idxp, table, out, idx_v, rows_v, sem_g, sem_o):
        c = lax.axis_index("c")
        s = lax.axis_index("s")
        wid = c * _NS + s
        base = wid * RG
        pltpu.sync_copy(idxp.at[pl.ds(base, RG)], idx_v)
        gs = [pltpu.async_copy(table.at[idx_v.at[j]], rows_v.at[j], sem_g)
              for j in range(RG)]
        for g in gs:
            g.wait()
        os_ = [pltpu.async_copy(rows_v.at[j],
                                out.at[pl.ds((base + j) * 128, 128)], sem_o)
               for j in range(RG)]
        for t in os_:
            t.wait()

    return k


_get_segsum = functools.cache(_make_segsum)
_get_knn_gather = functools.cache(_make_knn_gather)


def _segsum0(arr, y, zeros):
    return _get_segsum(NP0, H, R0)(arr, y, zeros)


def _segsum1(arr, y, zeros):
    return _get_segsum(NP1, H, R1)(arr, y, zeros)


def _segsum1s(arr, y, zeros):
    return _get_segsum(NP1, 16, R1)(arr, y, zeros)


def _knn_gather(idxp, table):
    return _get_knn_gather()(idxp, table)


def _pad_edges(e, EP, N):
    src = jnp.concatenate(
        [e[0].astype(jnp.int32), jnp.zeros((EP - e.shape[1],), jnp.int32)])
    dst = jnp.concatenate(
        [e[1].astype(jnp.int32),
         jnp.full((EP - e.shape[1],), N, jnp.int32)])
    return jnp.stack([src.reshape(-1, 128), dst.reshape(-1, 128)], axis=1)


def _row(v):
    return v.reshape(1, -1)


def kernel(latent, pos_0, pos_1, edge_index_0, edge_index_1, lin_W, lin_b,
           bn_g0, bn_b0, bn_g1, bn_b1, bn_g2, bn_b2, bn_g3, bn_b3,
           Wr0, br0, Wroot0, Wr1, br1, Wroot1, Wr2, br2, Wroot2,
           Wr3, br3, Wroot3):
    f32 = jnp.float32
    arr0 = _pad_edges(edge_index_0, EP0, N0)
    arr1 = _pad_edges(edge_index_1, EP1, N1)
    zeros0 = jnp.zeros((NP0, H), f32)
    zeros1 = jnp.zeros((NP1, H), f32)
    zeros1s = jnp.zeros((NP1, 16), f32)

    wr3x = jnp.pad(Wr3[:H], ((0, 0), (0, 16 - OUT)))
    wr3p = jnp.pad(Wr3[H:], ((0, 0), (0, 16 - OUT)))
    wt3x = jnp.pad(Wroot3[:H], ((0, 0), (0, 16 - OUT)))
    wt3p = jnp.pad(Wroot3[H:], ((0, 0), (0, 16 - OUT)))
    br3p = _row(jnp.pad(br3, (0, 16 - OUT)))

    y1, r1 = _call(
        _stage_a_body, [_sds((N0, H)), _sds((N0, H))])(
        latent, lin_W, _row(lin_b), _row(bn_g0), _row(bn_b0), pos_0,
        Wr0[:H], Wr0[H:], Wroot0[:H], Wroot0[H:], _row(br0))

    agg1 = _segsum0(arr0, y1, zeros0)[:, :N0]

    y2, r2 = _call(
        _stage_mid_body, [_sds((N0, H)), _sds((N0, H))])(
        agg1, r1, _row(bn_g1), _row(bn_b1), pos_0,
        Wr1[:H], Wr1[H:], Wroot1[:H], Wroot1[H:], _row(br1))

    agg2 = _segsum0(arr0, y2, zeros0)[:, :N0]

    x2 = _call(_stage_x_body, _sds((N0, H)))(
        agg2, r2, _row(bn_g2), _row(bn_b2))

    pos1p = jnp.pad(pos_1, ((0, NQP - N1), (0, 0)))
    pos0t = jnp.pad(pos_0, ((0, CP0 - N0), (0, 0))).T
    idx8, w8 = _call(
        _knn_body,
        [_sds((NQP, 8), jnp.int32), _sds((NQP, 8), f32)],
        grid=(NQP // KNN_BQ,),
        in_specs=[pl.BlockSpec((KNN_BQ, 3), lambda i: (i, 0)),
                  pl.BlockSpec((3, CP0), lambda i: (0, 0))],
        out_specs=[pl.BlockSpec((KNN_BQ, 8), lambda i: (i, 0)),
                   pl.BlockSpec((KNN_BQ, 8), lambda i: (i, 0))],
    )(pos1p, pos0t)

    idx_flat = jnp.pad(idx8[:N1, :3].reshape(-1), (0, GQ - 3 * N1))
    g = _knn_gather(idx_flat.reshape(-1, 128), x2)
    g3 = g[:3 * N1].reshape(N1, 3, H)

    y3, r3 = _call(
        _stage_interp_body, [_sds((N1, H)), _sds((N1, H))])(
        g3[:, 0], g3[:, 1], g3[:, 2], w8[:N1], pos_1,
        Wr2[:H], Wr2[H:], Wroot2[:H], Wroot2[H:], _row(br2))

    agg3 = _segsum1(arr1, y3, zeros1)[:, :N1]

    y4, r4 = _call(
        _stage_mid_body, [_sds((N1, 16)), _sds((N1, 16))])(
        agg3, r3, _row(bn_g3), _row(bn_b3), pos_1,
        wr3x, wr3p, wt3x, wt3p, br3p)

    agg4 = _segsum1s(arr1, y4, zeros1s)[:, :N1]

    out = _call(_stage_fin_body, _sds((N1, OUT)))(agg4, r4)
    return out


# speedup vs baseline: 1.5843x; 1.5843x over previous
import functools
import math

import jax
import jax.numpy as jnp
from jax import lax
from jax.experimental import pallas as pl
from jax.experimental.pallas import tpu as pltpu
from jax.experimental.pallas import tpu_sc as plsc

_TAN30 = math.tan(math.pi / 6)
_SCALE = 1.0 / 0.56 - 1.0

N0, N1 = 5000, 10000
E0, E1 = 320000, 640000
H, LDIM, D, OUT = 64, 32, 3, 4
NP0, NP1 = 5152, 10144
EP0, EP1 = 323584, 643072
R0 = EP0 // 128 // 32
R1 = EP1 // 128 // 32
CCH = 2
CP0 = 5120
NQP = 10240
KNN_BQ = 512
GQ = 32768

_NS = 16


def _elu(x):
    return jnp.where(x > 0, x, jnp.exp(jnp.minimum(x, 0.0)) - 1.0)


def _bn(z, g, b):
    m = jnp.mean(z, axis=0, keepdims=True)
    zc = z - m
    v = jnp.mean(zc * zc, axis=0, keepdims=True)
    return zc * lax.rsqrt(v + 1e-5) * g + b


def _dot(a, w):
    return jnp.dot(a, w, preferred_element_type=jnp.float32)


def _tail(x, pos, wrx, wrp, wtx, wtp, br, y_o, r_o):
    y_o[...] = _dot(x, wrx) + _dot(pos, wrp)
    r_o[...] = _dot(x, wtx) + _dot(pos, wtp) + br


def _stage_a_body(lat, lW, lb, g, b, pos, wrx, wrp, wtx, wtp, br, y_o, r_o):
    z = _dot(lat[...], lW[...]) + lb[...]
    x = _elu(_bn(z, g[...], b[...]))
    _tail(x, pos[...], wrx[...], wrp[...], wtx[...], wtp[...], br[...], y_o, r_o)


def _stage_mid_body(agg, r, g, b, pos, wrx, wrp, wtx, wtp, br, y_o, r_o):
    z = agg[0] + agg[1] + r[...]
    x = _elu(_bn(z, g[...], b[...]))
    _tail(x, pos[...], wrx[...], wrp[...], wtx[...], wtp[...], br[...], y_o, r_o)


def _stage_x_body(agg, r, g, b, x_o):
    z = agg[0] + agg[1] + r[...]
    x_o[...] = _elu(_bn(z, g[...], b[...]))


def _stage_interp_body(g0, g1, g2, w, pos, wrx, wrp, wtx, wtp, br,
                       y_o, r_o):
    w0 = w[:, 0:1]
    w1 = w[:, 1:2]
    w2 = w[:, 2:3]
    x = (g0[...] * w0 + g1[...] * w1 + g2[...] * w2) / (w0 + w1 + w2)
    _tail(x, pos[...], wrx[...], wrp[...], wtx[...], wtp[...], br[...], y_o, r_o)


def _stage_fin_body(agg, r, o):
    o[...] = (agg[0] + agg[1] + r[...])[:, :OUT]


def _transform_rows(px, py, pz):
    nx = px - _TAN30 * py
    s = 1.0 + _SCALE * (py / 1.1963)
    return nx * s, py * s, pz * s


def _knn_body(q_ref, pt_ref, idx_o, w_o):
    q = q_ref[...]
    qx = q[:, 0:1] - _TAN30 * q[:, 1:2]
    qs = 1.0 + _SCALE * (q[:, 1:2] / 1.1963)
    tqx, tqy, tqz = qx * qs, q[:, 1:2] * qs, q[:, 2:3] * qs
    qn = tqx * tqx + tqy * tqy + tqz * tqz

    tx, ty, tz = _transform_rows(pt_ref[0:1, :], pt_ref[1:2, :], pt_ref[2:3, :])
    cn = tx * tx + ty * ty + tz * tz
    colr = lax.broadcasted_iota(jnp.int32, (1, CP0), 1)
    cn = jnp.where(colr < N0, cn, jnp.inf)
    tp = jnp.concatenate([tx, ty, tz], axis=0)
    tq = jnp.concatenate([tqx, tqy, tqz], axis=1)
    d = qn + cn - 2.0 * _dot(tq, tp)

    col = lax.broadcasted_iota(jnp.int32, (KNN_BQ, CP0), 1)
    cols = []
    for _ in range(3):
        m = jnp.min(d, axis=1, keepdims=True)
        i = jnp.min(jnp.where(d == m, col, jnp.int32(2**30)), axis=1,
                    keepdims=True)
        cols.append((i, 1.0 / jnp.clip(m, 1e-16, None)))
        d = jnp.where(col == i, jnp.inf, d)
    z = jnp.zeros_like(cols[0][0])
    idx_o[...] = jnp.concatenate(
        [cols[0][0], cols[1][0], cols[2][0], z, z, z, z, z], axis=1)
    zf = jnp.zeros_like(cols[0][1])
    w_o[...] = jnp.concatenate(
        [cols[0][1], cols[1][1], cols[2][1], zf, zf, zf, zf, zf], axis=1)


def _sds(shape, dtype=jnp.float32):
    return jax.ShapeDtypeStruct(shape, dtype)


def _call(body, out_shape, **kw):
    kw.setdefault("compiler_params",
                  pltpu.CompilerParams(vmem_limit_bytes=100 * 1024 * 1024))
    return pl.pallas_call(body, out_shape=out_shape, **kw)


def _make_segsum(NP, W, R):
    mesh = plsc.VectorSubcoreMesh(core_axis_name="c", subcore_axis_name="s")
    sl = NP // _NS

    @functools.partial(
        pl.kernel, mesh=mesh,
        out_type=_sds((2, NP, W)),
        compiler_params=pltpu.CompilerParams(use_tc_tiling_on_sc=False),
        scratch_types=[
            pltpu.VMEM((2, 2, 128), jnp.int32),
            pltpu.VMEM((2, 128, W), jnp.float32),
            pltpu.VMEM_SHARED((NP, W), jnp.float32),
            pltpu.SemaphoreType.DMA,
        ],
    )
    def k(edges, y, zeros, out, idx_v, rows_v, acc, sem_g):
        c = lax.axis_index("c")
        s = lax.axis_index("s")
        wid = c * _NS + s
        base = wid * R
        pltpu.sync_copy(zeros.at[pl.ds(s * sl, sl)], acc.at[pl.ds(s * sl, sl)])
        plsc.subcore_barrier()

        def body(g, carry):
            pltpu.sync_copy(edges.at[base + g], idx_v.at[0])
            pltpu.async_copy(y.at[idx_v.at[0, 0]], rows_v.at[0], sem_g).wait()
            pltpu.sync_copy(rows_v.at[0], acc.at[idx_v.at[0, 1]], add=True)
            return carry

        lax.fori_loop(0, R, body, 0)
        plsc.subcore_barrier()
        pltpu.sync_copy(acc.at[pl.ds(s * sl, sl)], out.at[c, pl.ds(s * sl, sl)])

    return k


def _make_knn_gather():
    mesh = plsc.VectorSubcoreMesh(core_axis_name="c", subcore_axis_name="s")
    RG = GQ // 128 // 32

    @functools.partial(
        pl.kernel, mesh=mesh,
        out_type=_sds((GQ, H)),
        compiler_params=pltpu.CompilerParams(use_tc_tiling_on_sc=False),
        scratch_types=[
            pltpu.VMEM((RG, 128), jnp.int32),
            pltpu.VMEM((RG, 128, H), jnp.float32),
            pltpu.SemaphoreType.DMA,
            pltpu.SemaphoreType.DMA,
        ],
    )
    def k(idxp, table, out, idx_v, rows_v, sem_g, sem_o):
        c = lax.axis_index("c")
        s = lax.axis_index("s")
        wid = c * _NS + s
        base = wid * RG
        pltpu.sync_copy(idxp.at[pl.ds(base, RG)], idx_v)
        gs = [pltpu.async_copy(table.at[idx_v.at[j]], rows_v.at[j], sem_g)
              for j in range(RG)]
        for g in gs:
            g.wait()
        os_ = [pltpu.async_copy(rows_v.at[j],
                                out.at[pl.ds((base + j) * 128, 128)], sem_o)
               for j in range(RG)]
        for t in os_:
            t.wait()

    return k


_get_segsum = functools.cache(_make_segsum)
_get_knn_gather = functools.cache(_make_knn_gather)


def _segsum0(arr, y, zeros):
    return _get_segsum(NP0, H, R0)(arr, y, zeros)


def _segsum1(arr, y, zeros):
    return _get_segsum(NP1, H, R1)(arr, y, zeros)


def _segsum1s(arr, y, zeros):
    return _get_segsum(NP1, 16, R1)(arr, y, zeros)


def _knn_gather(idxp, table):
    return _get_knn_gather()(idxp, table)


def _pad_edges(e, EP, N):
    npad = EP - e.shape[1]
    cyc = jnp.tile(jnp.arange(128, dtype=jnp.int32), npad // 128 + 1)[:npad]
    src = jnp.concatenate([e[0].astype(jnp.int32), cyc])
    dst = jnp.concatenate([e[1].astype(jnp.int32), cyc + N])
    return jnp.stack([src.reshape(-1, 128), dst.reshape(-1, 128)], axis=1)


def _row(v):
    return v.reshape(1, -1)


def kernel(latent, pos_0, pos_1, edge_index_0, edge_index_1, lin_W, lin_b,
           bn_g0, bn_b0, bn_g1, bn_b1, bn_g2, bn_b2, bn_g3, bn_b3,
           Wr0, br0, Wroot0, Wr1, br1, Wroot1, Wr2, br2, Wroot2,
           Wr3, br3, Wroot3):
    f32 = jnp.float32
    arr0 = _pad_edges(edge_index_0, EP0, N0)
    arr1 = _pad_edges(edge_index_1, EP1, N1)
    zeros0 = jnp.zeros((NP0, H), f32)
    zeros1 = jnp.zeros((NP1, H), f32)
    zeros1s = jnp.zeros((NP1, 16), f32)

    wr3x = jnp.pad(Wr3[:H], ((0, 0), (0, 16 - OUT)))
    wr3p = jnp.pad(Wr3[H:], ((0, 0), (0, 16 - OUT)))
    wt3x = jnp.pad(Wroot3[:H], ((0, 0), (0, 16 - OUT)))
    wt3p = jnp.pad(Wroot3[H:], ((0, 0), (0, 16 - OUT)))
    br3p = _row(jnp.pad(br3, (0, 16 - OUT)))

    y1, r1 = _call(
        _stage_a_body, [_sds((N0, H)), _sds((N0, H))])(
        latent, lin_W, _row(lin_b), _row(bn_g0), _row(bn_b0), pos_0,
        Wr0[:H], Wr0[H:], Wroot0[:H], Wroot0[H:], _row(br0))

    agg1 = _segsum0(arr0, y1, zeros0)[:, :N0]

    y2, r2 = _call(
        _stage_mid_body, [_sds((N0, H)), _sds((N0, H))])(
        agg1, r1, _row(bn_g1), _row(bn_b1), pos_0,
        Wr1[:H], Wr1[H:], Wroot1[:H], Wroot1[H:], _row(br1))

    agg2 = _segsum0(arr0, y2, zeros0)[:, :N0]

    x2 = _call(_stage_x_body, _sds((N0, H)))(
        agg2, r2, _row(bn_g2), _row(bn_b2))

    pos1p = jnp.pad(pos_1, ((0, NQP - N1), (0, 0)))
    pos0t = jnp.pad(pos_0, ((0, CP0 - N0), (0, 0))).T
    idx8, w8 = _call(
        _knn_body,
        [_sds((NQP, 8), jnp.int32), _sds((NQP, 8), f32)],
        grid=(NQP // KNN_BQ,),
        in_specs=[pl.BlockSpec((KNN_BQ, 3), lambda i: (i, 0)),
                  pl.BlockSpec((3, CP0), lambda i: (0, 0))],
        out_specs=[pl.BlockSpec((KNN_BQ, 8), lambda i: (i, 0)),
                   pl.BlockSpec((KNN_BQ, 8), lambda i: (i, 0))],
    )(pos1p, pos0t)

    idx_flat = jnp.pad(idx8[:N1, :3].reshape(-1), (0, GQ - 3 * N1))
    g = _knn_gather(idx_flat.reshape(-1, 128), x2)
    g3 = g[:3 * N1].reshape(N1, 3, H)

    y3, r3 = _call(
        _stage_interp_body, [_sds((N1, H)), _sds((N1, H))])(
        g3[:, 0], g3[:, 1], g3[:, 2], w8[:N1], pos_1,
        Wr2[:H], Wr2[H:], Wroot2[:H], Wroot2[H:], _row(br2))

    agg3 = _segsum1(arr1, y3, zeros1)[:, :N1]

    y4, r4 = _call(
        _stage_mid_body, [_sds((N1, 16)), _sds((N1, 16))])(
        agg3, r3, _row(bn_g3), _row(bn_b3), pos_1,
        wr3x, wr3p, wt3x, wt3p, br3p)

    agg4 = _segsum1s(arr1, y4, zeros1s)[:, :N1]

    out = _call(_stage_fin_body, _sds((N1, OUT)))(agg4, r4)
    return out


# speedup vs baseline: 2.1432x; 1.3527x over previous
import functools
import math

import jax
import jax.numpy as jnp
from jax import lax
from jax.experimental import pallas as pl
from jax.experimental.pallas import tpu as pltpu
from jax.experimental.pallas import tpu_sc as plsc

_TAN30 = math.tan(math.pi / 6)
_SCALE = 1.0 / 0.56 - 1.0

N0, N1 = 5000, 10000
E0, E1 = 320000, 640000
H, LDIM, D, OUT = 64, 32, 3, 4
NP0, NP1 = 5152, 10144
EP0, EP1 = 327680, 655360
R0 = EP0 // 128 // 32
R1 = EP1 // 128 // 32
CCH = 8
CCH = 2
CP0 = 5120
NQP = 10240
KNN_BQ = 512
GQ = 32768

_NS = 16


def _elu(x):
    return jnp.where(x > 0, x, jnp.exp(jnp.minimum(x, 0.0)) - 1.0)


def _bn(z, g, b):
    m = jnp.mean(z, axis=0, keepdims=True)
    zc = z - m
    v = jnp.mean(zc * zc, axis=0, keepdims=True)
    return zc * lax.rsqrt(v + 1e-5) * g + b


def _dot(a, w):
    return jnp.dot(a, w, preferred_element_type=jnp.float32)


def _tail(x, pos, wrx, wrp, wtx, wtp, br, y_o, r_o):
    y_o[...] = _dot(x, wrx) + _dot(pos, wrp)
    r_o[...] = _dot(x, wtx) + _dot(pos, wtp) + br


def _stage_a_body(lat, lW, lb, g, b, pos, wrx, wrp, wtx, wtp, br, y_o, r_o):
    z = _dot(lat[...], lW[...]) + lb[...]
    x = _elu(_bn(z, g[...], b[...]))
    _tail(x, pos[...], wrx[...], wrp[...], wtx[...], wtp[...], br[...], y_o, r_o)


def _stage_mid_body(agg, r, g, b, pos, wrx, wrp, wtx, wtp, br, y_o, r_o):
    z = agg[0] + agg[1] + r[...]
    x = _elu(_bn(z, g[...], b[...]))
    _tail(x, pos[...], wrx[...], wrp[...], wtx[...], wtp[...], br[...], y_o, r_o)


def _stage_x_body(agg, r, g, b, x_o):
    z = agg[0] + agg[1] + r[...]
    x_o[...] = _elu(_bn(z, g[...], b[...]))


def _stage_interp_body(g0, g1, g2, w, pos, wrx, wrp, wtx, wtp, br,
                       y_o, r_o):
    w0 = w[:, 0:1]
    w1 = w[:, 1:2]
    w2 = w[:, 2:3]
    x = (g0[...] * w0 + g1[...] * w1 + g2[...] * w2) / (w0 + w1 + w2)
    _tail(x, pos[...], wrx[...], wrp[...], wtx[...], wtp[...], br[...], y_o, r_o)


def _stage_fin_body(agg, r, o):
    o[...] = (agg[0] + agg[1] + r[...])[:, :OUT]


def _transform_rows(px, py, pz):
    nx = px - _TAN30 * py
    s = 1.0 + _SCALE * (py / 1.1963)
    return nx * s, py * s, pz * s


def _knn_body(q_ref, pt_ref, idx_o, w_o):
    q = q_ref[...]
    qx = q[:, 0:1] - _TAN30 * q[:, 1:2]
    qs = 1.0 + _SCALE * (q[:, 1:2] / 1.1963)
    tqx, tqy, tqz = qx * qs, q[:, 1:2] * qs, q[:, 2:3] * qs
    qn = tqx * tqx + tqy * tqy + tqz * tqz

    tx, ty, tz = _transform_rows(pt_ref[0:1, :], pt_ref[1:2, :], pt_ref[2:3, :])
    cn = tx * tx + ty * ty + tz * tz
    colr = lax.broadcasted_iota(jnp.int32, (1, CP0), 1)
    cn = jnp.where(colr < N0, cn, jnp.inf)
    tp = jnp.concatenate([tx, ty, tz], axis=0)
    tq = jnp.concatenate([tqx, tqy, tqz], axis=1)
    d = qn + cn - 2.0 * _dot(tq, tp)

    col = lax.broadcasted_iota(jnp.int32, (KNN_BQ, CP0), 1)
    cols = []
    for _ in range(3):
        m = jnp.min(d, axis=1, keepdims=True)
        i = jnp.min(jnp.where(d == m, col, jnp.int32(2**30)), axis=1,
                    keepdims=True)
        cols.append((i, 1.0 / jnp.clip(m, 1e-16, None)))
        d = jnp.where(col == i, jnp.inf, d)
    z = jnp.zeros_like(cols[0][0])
    idx_o[...] = jnp.concatenate(
        [cols[0][0], cols[1][0], cols[2][0], z, z, z, z, z], axis=1)
    zf = jnp.zeros_like(cols[0][1])
    w_o[...] = jnp.concatenate(
        [cols[0][1], cols[1][1], cols[2][1], zf, zf, zf, zf, zf], axis=1)


def _sds(shape, dtype=jnp.float32):
    return jax.ShapeDtypeStruct(shape, dtype)


def _call(body, out_shape, **kw):
    kw.setdefault("compiler_params",
                  pltpu.CompilerParams(vmem_limit_bytes=100 * 1024 * 1024))
    return pl.pallas_call(body, out_shape=out_shape, **kw)


def _make_segsum(NP, W, R):
    mesh = plsc.VectorSubcoreMesh(core_axis_name="c", subcore_axis_name="s")
    sl = NP // _NS

    C = CCH
    nch = R // C

    @functools.partial(
        pl.kernel, mesh=mesh,
        out_type=_sds((2, NP, W)),
        compiler_params=pltpu.CompilerParams(use_tc_tiling_on_sc=False),
        scratch_types=[
            pltpu.VMEM((2, C, 2, 128), jnp.int32),
            pltpu.VMEM((C, 128, W), jnp.float32),
            pltpu.VMEM_SHARED((NP, W), jnp.float32),
            pltpu.SemaphoreType.DMA,
            pltpu.SemaphoreType.DMA,
            pltpu.SemaphoreType.DMA,
        ],
    )
    def k(edges, y, zeros, out, idx_v, rows_v, acc, sem_i, sem_g, sem_s):
        c = lax.axis_index("c")
        s = lax.axis_index("s")
        wid = c * _NS + s
        base = wid * R
        ip = pltpu.async_copy(edges.at[pl.ds(base, C)], idx_v.at[0], sem_i)
        pltpu.sync_copy(zeros.at[pl.ds(s * sl, sl)], acc.at[pl.ds(s * sl, sl)])
        plsc.subcore_barrier()
        ip.wait()

        def chunk(ch, carry):
            cur = lax.rem(ch, 2)
            nxt = 1 - cur
            nb = jnp.where(ch + 1 < nch, base + (ch + 1) * C, base)
            pltpu.async_copy(edges.at[pl.ds(nb, C)], idx_v.at[nxt], sem_i)
            gs = [pltpu.async_copy(y.at[idx_v.at[cur, b, 0]], rows_v.at[b],
                                   sem_g) for b in range(C)]
            for g in gs:
                g.wait()
            ss = [pltpu.async_copy(rows_v.at[b], acc.at[idx_v.at[cur, b, 1]],
                                   sem_s, add=True) for b in range(C)]
            for t in ss:
                t.wait()
            pltpu.make_async_copy(edges.at[pl.ds(base, C)], idx_v.at[nxt],
                                  sem_i).wait()
            return carry

        lax.fori_loop(0, nch, chunk, 0)
        plsc.subcore_barrier()
        pltpu.sync_copy(acc.at[pl.ds(s * sl, sl)], out.at[c, pl.ds(s * sl, sl)])

    return k


def _make_knn_gather():
    mesh = plsc.VectorSubcoreMesh(core_axis_name="c", subcore_axis_name="s")
    RG = GQ // 128 // 32

    @functools.partial(
        pl.kernel, mesh=mesh,
        out_type=_sds((GQ, H)),
        compiler_params=pltpu.CompilerParams(use_tc_tiling_on_sc=False),
        scratch_types=[
            pltpu.VMEM((RG, 128), jnp.int32),
            pltpu.VMEM((RG, 128, H), jnp.float32),
            pltpu.SemaphoreType.DMA,
            pltpu.SemaphoreType.DMA,
        ],
    )
    def k(idxp, table, out, idx_v, rows_v, sem_g, sem_o):
        c = lax.axis_index("c")
        s = lax.axis_index("s")
        wid = c * _NS + s
        base = wid * RG
        pltpu.sync_copy(idxp.at[pl.ds(base, RG)], idx_v)
        gs = [pltpu.async_copy(table.at[idx_v.at[j]], rows_v.at[j], sem_g)
              for j in range(RG)]
        for g in gs:
            g.wait()
        os_ = [pltpu.async_copy(rows_v.at[j],
                                out.at[pl.ds((base + j) * 128, 128)], sem_o)
               for j in range(RG)]
        for t in os_:
            t.wait()

    return k


_get_segsum = functools.cache(_make_segsum)
_get_knn_gather = functools.cache(_make_knn_gather)


def _segsum0(arr, y, zeros):
    return _get_segsum(NP0, H, R0)(arr, y, zeros)


def _segsum1(arr, y, zeros):
    return _get_segsum(NP1, H, R1)(arr, y, zeros)


def _segsum1s(arr, y, zeros):
    return _get_segsum(NP1, 16, R1)(arr, y, zeros)


def _knn_gather(idxp, table):
    return _get_knn_gather()(idxp, table)


def _pad_edges(e, EP, N):
    npad = EP - e.shape[1]
    cyc = jnp.tile(jnp.arange(128, dtype=jnp.int32), npad // 128 + 1)[:npad]
    src = jnp.concatenate([e[0].astype(jnp.int32), cyc])
    dst = jnp.concatenate([e[1].astype(jnp.int32), cyc + N])
    return jnp.stack([src.reshape(-1, 128), dst.reshape(-1, 128)], axis=1)


def _row(v):
    return v.reshape(1, -1)


def kernel(latent, pos_0, pos_1, edge_index_0, edge_index_1, lin_W, lin_b,
           bn_g0, bn_b0, bn_g1, bn_b1, bn_g2, bn_b2, bn_g3, bn_b3,
           Wr0, br0, Wroot0, Wr1, br1, Wroot1, Wr2, br2, Wroot2,
           Wr3, br3, Wroot3):
    f32 = jnp.float32
    arr0 = _pad_edges(edge_index_0, EP0, N0)
    arr1 = _pad_edges(edge_index_1, EP1, N1)
    zeros0 = jnp.zeros((NP0, H), f32)
    zeros1 = jnp.zeros((NP1, H), f32)
    zeros1s = jnp.zeros((NP1, 16), f32)

    wr3x = jnp.pad(Wr3[:H], ((0, 0), (0, 16 - OUT)))
    wr3p = jnp.pad(Wr3[H:], ((0, 0), (0, 16 - OUT)))
    wt3x = jnp.pad(Wroot3[:H], ((0, 0), (0, 16 - OUT)))
    wt3p = jnp.pad(Wroot3[H:], ((0, 0), (0, 16 - OUT)))
    br3p = _row(jnp.pad(br3, (0, 16 - OUT)))

    y1, r1 = _call(
        _stage_a_body, [_sds((N0, H)), _sds((N0, H))])(
        latent, lin_W, _row(lin_b), _row(bn_g0), _row(bn_b0), pos_0,
        Wr0[:H], Wr0[H:], Wroot0[:H], Wroot0[H:], _row(br0))

    agg1 = _segsum0(arr0, y1, zeros0)[:, :N0]

    y2, r2 = _call(
        _stage_mid_body, [_sds((N0, H)), _sds((N0, H))])(
        agg1, r1, _row(bn_g1), _row(bn_b1), pos_0,
        Wr1[:H], Wr1[H:], Wroot1[:H], Wroot1[H:], _row(br1))

    agg2 = _segsum0(arr0, y2, zeros0)[:, :N0]

    x2 = _call(_stage_x_body, _sds((N0, H)))(
        agg2, r2, _row(bn_g2), _row(bn_b2))

    pos1p = jnp.pad(pos_1, ((0, NQP - N1), (0, 0)))
    pos0t = jnp.pad(pos_0, ((0, CP0 - N0), (0, 0))).T
    idx8, w8 = _call(
        _knn_body,
        [_sds((NQP, 8), jnp.int32), _sds((NQP, 8), f32)],
        grid=(NQP // KNN_BQ,),
        in_specs=[pl.BlockSpec((KNN_BQ, 3), lambda i: (i, 0)),
                  pl.BlockSpec((3, CP0), lambda i: (0, 0))],
        out_specs=[pl.BlockSpec((KNN_BQ, 8), lambda i: (i, 0)),
                   pl.BlockSpec((KNN_BQ, 8), lambda i: (i, 0))],
    )(pos1p, pos0t)

    idx_flat = jnp.pad(idx8[:N1, :3].reshape(-1), (0, GQ - 3 * N1))
    g = _knn_gather(idx_flat.reshape(-1, 128), x2)
    g3 = g[:3 * N1].reshape(N1, 3, H)

    y3, r3 = _call(
        _stage_interp_body, [_sds((N1, H)), _sds((N1, H))])(
        g3[:, 0], g3[:, 1], g3[:, 2], w8[:N1], pos_1,
        Wr2[:H], Wr2[H:], Wroot2[:H], Wroot2[H:], _row(br2))

    agg3 = _segsum1(arr1, y3, zeros1)[:, :N1]

    y4, r4 = _call(
        _stage_mid_body, [_sds((N1, 16)), _sds((N1, 16))])(
        agg3, r3, _row(bn_g3), _row(bn_b3), pos_1,
        wr3x, wr3p, wt3x, wt3p, br3p)

    agg4 = _segsum1s(arr1, y4, zeros1s)[:, :N1]

    out = _call(_stage_fin_body, _sds((N1, OUT)))(agg4, r4)
    return out


# speedup vs baseline: 2.2064x; 1.0295x over previous
import functools
import math

import jax
import jax.numpy as jnp
from jax import lax
from jax.experimental import pallas as pl
from jax.experimental.pallas import tpu as pltpu
from jax.experimental.pallas import tpu_sc as plsc

_TAN30 = math.tan(math.pi / 6)
_SCALE = 1.0 / 0.56 - 1.0

N0, N1 = 5000, 10000
E0, E1 = 320000, 640000
H, LDIM, D, OUT = 64, 32, 3, 4
NP0, NP1 = 5152, 10144
EP0, EP1 = 327680, 655360
R0 = EP0 // 128 // 32
R1 = EP1 // 128 // 32
CCH = 8
CCH = 2
CP0 = 5120
NQP = 10240
KNN_BQ = 512
GQ = 32768

_NS = 16


def _elu(x):
    return jnp.where(x > 0, x, jnp.exp(jnp.minimum(x, 0.0)) - 1.0)


def _bn(z, g, b):
    m = jnp.mean(z, axis=0, keepdims=True)
    zc = z - m
    v = jnp.mean(zc * zc, axis=0, keepdims=True)
    return zc * lax.rsqrt(v + 1e-5) * g + b


def _dot(a, w):
    return jnp.dot(a, w, preferred_element_type=jnp.float32)


def _tail(x, pos, wrx, wrp, wtx, wtp, br, y_o, r_o):
    y_o[...] = _dot(x, wrx) + _dot(pos, wrp)
    r_o[...] = _dot(x, wtx) + _dot(pos, wtp) + br


def _stage_a_body(lat, lW, lb, g, b, pos, wrx, wrp, wtx, wtp, br, y_o, r_o):
    z = _dot(lat[...], lW[...]) + lb[...]
    x = _elu(_bn(z, g[...], b[...]))
    _tail(x, pos[...], wrx[...], wrp[...], wtx[...], wtp[...], br[...], y_o, r_o)


def _stage_mid_body(agg, r, g, b, pos, wrx, wrp, wtx, wtp, br, y_o, r_o):
    z = agg[0] + agg[1] + r[...]
    x = _elu(_bn(z, g[...], b[...]))
    _tail(x, pos[...], wrx[...], wrp[...], wtx[...], wtp[...], br[...], y_o, r_o)


def _stage_x_body(agg, r, g, b, x_o):
    z = agg[0] + agg[1] + r[...]
    x_o[...] = _elu(_bn(z, g[...], b[...]))


def _stage_interp_body(g0, g1, g2, w, pos, wrx, wrp, wtx, wtp, br,
                       y_o, r_o):
    w0 = w[:, 0:1]
    w1 = w[:, 1:2]
    w2 = w[:, 2:3]
    x = (g0[...] * w0 + g1[...] * w1 + g2[...] * w2) / (w0 + w1 + w2)
    _tail(x, pos[...], wrx[...], wrp[...], wtx[...], wtp[...], br[...], y_o, r_o)


def _stage_fin_body(agg, r, o):
    o[...] = (agg[0] + agg[1] + r[...])[:, :OUT]


def _transform_rows(px, py, pz):
    nx = px - _TAN30 * py
    s = 1.0 + _SCALE * (py / 1.1963)
    return nx * s, py * s, pz * s


def _knn_body(q_ref, pt_ref, idx_o, w_o):
    q = q_ref[...]
    qx = q[:, 0:1] - _TAN30 * q[:, 1:2]
    qs = 1.0 + _SCALE * (q[:, 1:2] / 1.1963)
    tqx, tqy, tqz = qx * qs, q[:, 1:2] * qs, q[:, 2:3] * qs
    qn = tqx * tqx + tqy * tqy + tqz * tqz

    tx, ty, tz = _transform_rows(pt_ref[0:1, :], pt_ref[1:2, :], pt_ref[2:3, :])
    cn = tx * tx + ty * ty + tz * tz
    colr = lax.broadcasted_iota(jnp.int32, (1, CP0), 1)
    cn = jnp.where(colr < N0, cn, jnp.inf)
    tp = jnp.concatenate([tx, ty, tz], axis=0)
    tq = jnp.concatenate([tqx, tqy, tqz], axis=1)
    d = qn + cn - 2.0 * _dot(tq, tp)

    col = lax.broadcasted_iota(jnp.int32, (KNN_BQ, CP0), 1)
    cols = []
    for _ in range(3):
        m = jnp.min(d, axis=1, keepdims=True)
        i = jnp.min(jnp.where(d == m, col, jnp.int32(2**30)), axis=1,
                    keepdims=True)
        cols.append((i, 1.0 / jnp.clip(m, 1e-16, None)))
        d = jnp.where(col == i, jnp.inf, d)
    z = jnp.zeros_like(cols[0][0])
    idx_o[...] = jnp.concatenate(
        [cols[0][0], cols[1][0], cols[2][0], z, z, z, z, z], axis=1)
    zf = jnp.zeros_like(cols[0][1])
    w_o[...] = jnp.concatenate(
        [cols[0][1], cols[1][1], cols[2][1], zf, zf, zf, zf, zf], axis=1)


def _sds(shape, dtype=jnp.float32):
    return jax.ShapeDtypeStruct(shape, dtype)


def _call(body, out_shape, **kw):
    kw.setdefault("compiler_params",
                  pltpu.CompilerParams(vmem_limit_bytes=100 * 1024 * 1024))
    return pl.pallas_call(body, out_shape=out_shape, **kw)


def _make_segsum(NP, W, R):
    mesh = plsc.VectorSubcoreMesh(core_axis_name="c", subcore_axis_name="s")
    sl = NP // _NS

    C = CCH if W >= 64 else 2 * CCH
    nch = R // C

    @functools.partial(
        pl.kernel, mesh=mesh,
        out_type=_sds((2, NP, W)),
        compiler_params=pltpu.CompilerParams(use_tc_tiling_on_sc=False),
        scratch_types=[
            pltpu.VMEM((2, C, 2, 128), jnp.int32),
            pltpu.VMEM((C, 128, W), jnp.float32),
            pltpu.VMEM_SHARED((NP, W), jnp.float32),
            pltpu.SemaphoreType.DMA,
            pltpu.SemaphoreType.DMA,
            pltpu.SemaphoreType.DMA,
        ],
    )
    def k(edges, y, zeros, out, idx_v, rows_v, acc, sem_i, sem_g, sem_s):
        c = lax.axis_index("c")
        s = lax.axis_index("s")
        wid = c * _NS + s
        base = wid * R
        ip = pltpu.async_copy(edges.at[pl.ds(base, C)], idx_v.at[0], sem_i)
        pltpu.sync_copy(zeros.at[pl.ds(s * sl, sl)], acc.at[pl.ds(s * sl, sl)])
        plsc.subcore_barrier()
        ip.wait()

        def chunk(ch, carry):
            cur = lax.rem(ch, 2)
            nxt = 1 - cur
            nb = jnp.where(ch + 1 < nch, base + (ch + 1) * C, base)
            pltpu.async_copy(edges.at[pl.ds(nb, C)], idx_v.at[nxt], sem_i)
            gs = [pltpu.async_copy(y.at[idx_v.at[cur, b, 0]], rows_v.at[b],
                                   sem_g) for b in range(C)]
            for g in gs:
                g.wait()
            ss = [pltpu.async_copy(rows_v.at[b], acc.at[idx_v.at[cur, b, 1]],
                                   sem_s, add=True) for b in range(C)]
            for t in ss:
                t.wait()
            pltpu.make_async_copy(edges.at[pl.ds(base, C)], idx_v.at[nxt],
                                  sem_i).wait()
            return carry

        lax.fori_loop(0, nch, chunk, 0)
        plsc.subcore_barrier()
        pltpu.sync_copy(acc.at[pl.ds(s * sl, sl)], out.at[c, pl.ds(s * sl, sl)])

    return k


def _make_knn_gather():
    mesh = plsc.VectorSubcoreMesh(core_axis_name="c", subcore_axis_name="s")
    RG = GQ // 128 // 32

    @functools.partial(
        pl.kernel, mesh=mesh,
        out_type=_sds((GQ, H)),
        compiler_params=pltpu.CompilerParams(use_tc_tiling_on_sc=False),
        scratch_types=[
            pltpu.VMEM((RG, 128), jnp.int32),
            pltpu.VMEM((RG, 128, H), jnp.float32),
            pltpu.SemaphoreType.DMA,
            pltpu.SemaphoreType.DMA,
        ],
    )
    def k(idxp, table, out, idx_v, rows_v, sem_g, sem_o):
        c = lax.axis_index("c")
        s = lax.axis_index("s")
        wid = c * _NS + s
        base = wid * RG
        pltpu.sync_copy(idxp.at[pl.ds(base, RG)], idx_v)
        gs = [pltpu.async_copy(table.at[idx_v.at[j]], rows_v.at[j], sem_g)
              for j in range(RG)]
        for g in gs:
            g.wait()
        os_ = [pltpu.async_copy(rows_v.at[j],
                                out.at[pl.ds((base + j) * 128, 128)], sem_o)
               for j in range(RG)]
        for t in os_:
            t.wait()

    return k


_get_segsum = functools.cache(_make_segsum)
_get_knn_gather = functools.cache(_make_knn_gather)


def _segsum0(arr, y, zeros):
    return _get_segsum(NP0, H, R0)(arr, y, zeros)


def _segsum1(arr, y, zeros):
    return _get_segsum(NP1, H, R1)(arr, y, zeros)


def _segsum1s(arr, y, zeros):
    return _get_segsum(NP1, 16, R1)(arr, y, zeros)


def _knn_gather(idxp, table):
    return _get_knn_gather()(idxp, table)


def _pad_edges(e, EP, N):
    npad = EP - e.shape[1]
    cyc = jnp.tile(jnp.arange(128, dtype=jnp.int32), npad // 128 + 1)[:npad]
    src = jnp.concatenate([e[0].astype(jnp.int32), cyc])
    dst = jnp.concatenate([e[1].astype(jnp.int32), cyc + N])
    return jnp.stack([src.reshape(-1, 128), dst.reshape(-1, 128)], axis=1)


def _row(v):
    return v.reshape(1, -1)


def kernel(latent, pos_0, pos_1, edge_index_0, edge_index_1, lin_W, lin_b,
           bn_g0, bn_b0, bn_g1, bn_b1, bn_g2, bn_b2, bn_g3, bn_b3,
           Wr0, br0, Wroot0, Wr1, br1, Wroot1, Wr2, br2, Wroot2,
           Wr3, br3, Wroot3):
    f32 = jnp.float32
    arr0 = _pad_edges(edge_index_0, EP0, N0)
    arr1 = _pad_edges(edge_index_1, EP1, N1)
    zeros0 = jnp.zeros((NP0, H), f32)
    zeros1 = jnp.zeros((NP1, H), f32)
    zeros1s = jnp.zeros((NP1, 16), f32)

    wr3x = jnp.pad(Wr3[:H], ((0, 0), (0, 16 - OUT)))
    wr3p = jnp.pad(Wr3[H:], ((0, 0), (0, 16 - OUT)))
    wt3x = jnp.pad(Wroot3[:H], ((0, 0), (0, 16 - OUT)))
    wt3p = jnp.pad(Wroot3[H:], ((0, 0), (0, 16 - OUT)))
    br3p = _row(jnp.pad(br3, (0, 16 - OUT)))

    y1, r1 = _call(
        _stage_a_body, [_sds((N0, H)), _sds((N0, H))])(
        latent, lin_W, _row(lin_b), _row(bn_g0), _row(bn_b0), pos_0,
        Wr0[:H], Wr0[H:], Wroot0[:H], Wroot0[H:], _row(br0))

    agg1 = _segsum0(arr0, y1, zeros0)[:, :N0]

    y2, r2 = _call(
        _stage_mid_body, [_sds((N0, H)), _sds((N0, H))])(
        agg1, r1, _row(bn_g1), _row(bn_b1), pos_0,
        Wr1[:H], Wr1[H:], Wroot1[:H], Wroot1[H:], _row(br1))

    agg2 = _segsum0(arr0, y2, zeros0)[:, :N0]

    x2 = _call(_stage_x_body, _sds((N0, H)))(
        agg2, r2, _row(bn_g2), _row(bn_b2))

    pos1p = jnp.pad(pos_1, ((0, NQP - N1), (0, 0)))
    pos0t = jnp.pad(pos_0, ((0, CP0 - N0), (0, 0))).T
    idx8, w8 = _call(
        _knn_body,
        [_sds((NQP, 8), jnp.int32), _sds((NQP, 8), f32)],
        grid=(NQP // KNN_BQ,),
        in_specs=[pl.BlockSpec((KNN_BQ, 3), lambda i: (i, 0)),
                  pl.BlockSpec((3, CP0), lambda i: (0, 0))],
        out_specs=[pl.BlockSpec((KNN_BQ, 8), lambda i: (i, 0)),
                   pl.BlockSpec((KNN_BQ, 8), lambda i: (i, 0))],
    )(pos1p, pos0t)

    idx_flat = jnp.pad(idx8[:N1, :3].reshape(-1), (0, GQ - 3 * N1))
    g = _knn_gather(idx_flat.reshape(-1, 128), x2)
    g3 = g[:3 * N1].reshape(N1, 3, H)

    y3, r3 = _call(
        _stage_interp_body, [_sds((N1, H)), _sds((N1, H))])(
        g3[:, 0], g3[:, 1], g3[:, 2], w8[:N1], pos_1,
        Wr2[:H], Wr2[H:], Wroot2[:H], Wroot2[H:], _row(br2))

    agg3 = _segsum1(arr1, y3, zeros1)[:, :N1]

    y4, r4 = _call(
        _stage_mid_body, [_sds((N1, 16)), _sds((N1, 16))])(
        agg3, r3, _row(bn_g3), _row(bn_b3), pos_1,
        wr3x, wr3p, wt3x, wt3p, br3p)

    agg4 = _segsum1s(arr1, y4, zeros1s)[:, :N1]

    out = _call(_stage_fin_body, _sds((N1, OUT)))(agg4, r4)
    return out


# speedup vs baseline: 2.3720x; 1.0751x over previous
import functools
import math

import jax
import jax.numpy as jnp
from jax import lax
from jax.experimental import pallas as pl
from jax.experimental.pallas import tpu as pltpu
from jax.experimental.pallas import tpu_sc as plsc

_TAN30 = math.tan(math.pi / 6)
_SCALE = 1.0 / 0.56 - 1.0

N0, N1 = 5000, 10000
E0, E1 = 320000, 640000
H, LDIM, D, OUT = 64, 32, 3, 4
NP0, NP1 = 5152, 10144
EP0, EP1 = 327680, 655360
R0 = EP0 // 128 // 32
R1 = EP1 // 128 // 32
CCH = 8
CCH = 2
CP0 = 5120
NQP = 10240
KNN_BQ = 512
GQ = 32768

_NS = 16


def _elu(x):
    return jnp.where(x > 0, x, jnp.exp(jnp.minimum(x, 0.0)) - 1.0)


def _bn(z, g, b):
    m = jnp.mean(z, axis=0, keepdims=True)
    zc = z - m
    v = jnp.mean(zc * zc, axis=0, keepdims=True)
    return zc * lax.rsqrt(v + 1e-5) * g + b


def _dot(a, w):
    return jnp.dot(a, w, preferred_element_type=jnp.float32)


def _tail(x, pos, wrx, wrp, wtx, wtp, br, y_o, r_o):
    y_o[...] = _dot(x, wrx) + _dot(pos, wrp)
    r_o[...] = _dot(x, wtx) + _dot(pos, wtp) + br


def _stage_a_body(lat, lW, lb, g, b, pos, wrx, wrp, wtx, wtp, br, y_o, r_o):
    z = _dot(lat[...], lW[...]) + lb[...]
    x = _elu(_bn(z, g[...], b[...]))
    _tail(x, pos[...], wrx[...], wrp[...], wtx[...], wtp[...], br[...], y_o, r_o)


def _stage_mid_body(agg, r, g, b, pos, wrx, wrp, wtx, wtp, br, y_o, r_o):
    z = agg[0] + agg[1] + r[...]
    x = _elu(_bn(z, g[...], b[...]))
    _tail(x, pos[...], wrx[...], wrp[...], wtx[...], wtp[...], br[...], y_o, r_o)


def _stage_x_body(agg, r, g, b, x_o):
    z = agg[0] + agg[1] + r[...]
    x_o[...] = _elu(_bn(z, g[...], b[...]))


def _stage_interp_body(g0, g1, g2, w, pos, wrx, wrp, wtx, wtp, br,
                       y_o, r_o):
    w0 = w[:, 0:1]
    w1 = w[:, 1:2]
    w2 = w[:, 2:3]
    x = (g0[...] * w0 + g1[...] * w1 + g2[...] * w2) / (w0 + w1 + w2)
    _tail(x, pos[...], wrx[...], wrp[...], wtx[...], wtp[...], br[...], y_o, r_o)


def _stage_fin_body(agg, r, o):
    o[...] = (agg[0] + agg[1] + r[...])[:, :OUT]


def _transform_rows(px, py, pz):
    nx = px - _TAN30 * py
    s = 1.0 + _SCALE * (py / 1.1963)
    return nx * s, py * s, pz * s


def _knn_body(q_ref, pt_ref, idx_o, w_o):
    q = q_ref[...]
    qx = q[:, 0:1] - _TAN30 * q[:, 1:2]
    qs = 1.0 + _SCALE * (q[:, 1:2] / 1.1963)
    tqx, tqy, tqz = qx * qs, q[:, 1:2] * qs, q[:, 2:3] * qs
    qn = tqx * tqx + tqy * tqy + tqz * tqz

    tx, ty, tz = _transform_rows(pt_ref[0:1, :], pt_ref[1:2, :], pt_ref[2:3, :])
    cn = tx * tx + ty * ty + tz * tz
    colr = lax.broadcasted_iota(jnp.int32, (1, CP0), 1)
    cn = jnp.where(colr < N0, cn, jnp.inf)
    tp = jnp.concatenate([tx, ty, tz], axis=0)
    tq = jnp.concatenate([tqx, tqy, tqz], axis=1)
    d = qn + cn - 2.0 * _dot(tq, tp)

    col = lax.broadcasted_iota(jnp.int32, (KNN_BQ, CP0), 1)
    cols = []
    for _ in range(3):
        m = jnp.min(d, axis=1, keepdims=True)
        i = jnp.min(jnp.where(d == m, col, jnp.int32(2**30)), axis=1,
                    keepdims=True)
        cols.append((i, 1.0 / jnp.clip(m, 1e-16, None)))
        d = jnp.where(col == i, jnp.inf, d)
    z = jnp.zeros_like(cols[0][0])
    idx_o[...] = jnp.concatenate(
        [cols[0][0], cols[1][0], cols[2][0], z, z, z, z, z], axis=1)
    zf = jnp.zeros_like(cols[0][1])
    w_o[...] = jnp.concatenate(
        [cols[0][1], cols[1][1], cols[2][1], zf, zf, zf, zf, zf], axis=1)


def _sds(shape, dtype=jnp.float32):
    return jax.ShapeDtypeStruct(shape, dtype)


def _call(body, out_shape, **kw):
    kw.setdefault("compiler_params",
                  pltpu.CompilerParams(vmem_limit_bytes=100 * 1024 * 1024))
    return pl.pallas_call(body, out_shape=out_shape, **kw)


def _make_segsum(NP, W, R):
    mesh = plsc.VectorSubcoreMesh(core_axis_name="c", subcore_axis_name="s")
    sl = NP // _NS

    C = 10 if W >= 64 else 20
    nch = R // C

    @functools.partial(
        pl.kernel, mesh=mesh,
        out_type=_sds((2, NP, W)),
        compiler_params=pltpu.CompilerParams(use_tc_tiling_on_sc=False),
        scratch_types=[
            pltpu.VMEM((2, C, 2, 128), jnp.int32),
            pltpu.VMEM((C, 128, W), jnp.float32),
            pltpu.VMEM_SHARED((NP, W), jnp.float32),
            pltpu.SemaphoreType.DMA,
            pltpu.SemaphoreType.DMA,
            pltpu.SemaphoreType.DMA,
        ],
    )
    def k(edges, y, zeros, out, idx_v, rows_v, acc, sem_i, sem_g, sem_s):
        c = lax.axis_index("c")
        s = lax.axis_index("s")
        wid = c * _NS + s
        base = wid * R
        ip = pltpu.async_copy(edges.at[pl.ds(base, C)], idx_v.at[0], sem_i)
        pltpu.sync_copy(zeros.at[pl.ds(s * sl, sl)], acc.at[pl.ds(s * sl, sl)])
        plsc.subcore_barrier()
        ip.wait()

        def chunk(ch, carry):
            cur = lax.rem(ch, 2)
            nxt = 1 - cur
            nb = jnp.where(ch + 1 < nch, base + (ch + 1) * C, base)
            pltpu.async_copy(edges.at[pl.ds(nb, C)], idx_v.at[nxt], sem_i)
            gs = [pltpu.async_copy(y.at[idx_v.at[cur, b, 0]], rows_v.at[b],
                                   sem_g) for b in range(C)]
            for g in gs:
                g.wait()
            ss = [pltpu.async_copy(rows_v.at[b], acc.at[idx_v.at[cur, b, 1]],
                                   sem_s, add=True) for b in range(C)]
            for t in ss:
                t.wait()
            pltpu.make_async_copy(edges.at[pl.ds(base, C)], idx_v.at[nxt],
                                  sem_i).wait()
            return carry

        lax.fori_loop(0, nch, chunk, 0)
        plsc.subcore_barrier()
        pltpu.sync_copy(acc.at[pl.ds(s * sl, sl)], out.at[c, pl.ds(s * sl, sl)])

    return k


def _make_knn_gather():
    mesh = plsc.VectorSubcoreMesh(core_axis_name="c", subcore_axis_name="s")
    RG = GQ // 128 // 32

    @functools.partial(
        pl.kernel, mesh=mesh,
        out_type=_sds((GQ, H)),
        compiler_params=pltpu.CompilerParams(use_tc_tiling_on_sc=False),
        scratch_types=[
            pltpu.VMEM((RG, 128), jnp.int32),
            pltpu.VMEM((RG, 128, H), jnp.float32),
            pltpu.SemaphoreType.DMA,
            pltpu.SemaphoreType.DMA,
        ],
    )
    def k(idxp, table, out, idx_v, rows_v, sem_g, sem_o):
        c = lax.axis_index("c")
        s = lax.axis_index("s")
        wid = c * _NS + s
        base = wid * RG
        pltpu.sync_copy(idxp.at[pl.ds(base, RG)], idx_v)
        gs = [pltpu.async_copy(table.at[idx_v.at[j]], rows_v.at[j], sem_g)
              for j in range(RG)]
        for g in gs:
            g.wait()
        os_ = [pltpu.async_copy(rows_v.at[j],
                                out.at[pl.ds((base + j) * 128, 128)], sem_o)
               for j in range(RG)]
        for t in os_:
            t.wait()

    return k


_get_segsum = functools.cache(_make_segsum)
_get_knn_gather = functools.cache(_make_knn_gather)


def _segsum0(arr, y, zeros):
    return _get_segsum(NP0, H, R0)(arr, y, zeros)


def _segsum1(arr, y, zeros):
    return _get_segsum(NP1, H, R1)(arr, y, zeros)


def _segsum1s(arr, y, zeros):
    return _get_segsum(NP1, 16, R1)(arr, y, zeros)


def _knn_gather(idxp, table):
    return _get_knn_gather()(idxp, table)


def _pad_edges(e, EP, N):
    npad = EP - e.shape[1]
    cyc = jnp.tile(jnp.arange(128, dtype=jnp.int32), npad // 128 + 1)[:npad]
    src = jnp.concatenate([e[0].astype(jnp.int32), cyc])
    dst = jnp.concatenate([e[1].astype(jnp.int32), cyc + N])
    return jnp.stack([src.reshape(-1, 128), dst.reshape(-1, 128)], axis=1)


def _row(v):
    return v.reshape(1, -1)


def kernel(latent, pos_0, pos_1, edge_index_0, edge_index_1, lin_W, lin_b,
           bn_g0, bn_b0, bn_g1, bn_b1, bn_g2, bn_b2, bn_g3, bn_b3,
           Wr0, br0, Wroot0, Wr1, br1, Wroot1, Wr2, br2, Wroot2,
           Wr3, br3, Wroot3):
    f32 = jnp.float32
    arr0 = _pad_edges(edge_index_0, EP0, N0)
    arr1 = _pad_edges(edge_index_1, EP1, N1)
    zeros0 = jnp.zeros((NP0, H), f32)
    zeros1 = jnp.zeros((NP1, H), f32)
    zeros1s = jnp.zeros((NP1, 16), f32)

    wr3x = jnp.pad(Wr3[:H], ((0, 0), (0, 16 - OUT)))
    wr3p = jnp.pad(Wr3[H:], ((0, 0), (0, 16 - OUT)))
    wt3x = jnp.pad(Wroot3[:H], ((0, 0), (0, 16 - OUT)))
    wt3p = jnp.pad(Wroot3[H:], ((0, 0), (0, 16 - OUT)))
    br3p = _row(jnp.pad(br3, (0, 16 - OUT)))

    y1, r1 = _call(
        _stage_a_body, [_sds((N0, H)), _sds((N0, H))])(
        latent, lin_W, _row(lin_b), _row(bn_g0), _row(bn_b0), pos_0,
        Wr0[:H], Wr0[H:], Wroot0[:H], Wroot0[H:], _row(br0))

    agg1 = _segsum0(arr0, y1, zeros0)[:, :N0]

    y2, r2 = _call(
        _stage_mid_body, [_sds((N0, H)), _sds((N0, H))])(
        agg1, r1, _row(bn_g1), _row(bn_b1), pos_0,
        Wr1[:H], Wr1[H:], Wroot1[:H], Wroot1[H:], _row(br1))

    agg2 = _segsum0(arr0, y2, zeros0)[:, :N0]

    x2 = _call(_stage_x_body, _sds((N0, H)))(
        agg2, r2, _row(bn_g2), _row(bn_b2))

    pos1p = jnp.pad(pos_1, ((0, NQP - N1), (0, 0)))
    pos0t = jnp.pad(pos_0, ((0, CP0 - N0), (0, 0))).T
    idx8, w8 = _call(
        _knn_body,
        [_sds((NQP, 8), jnp.int32), _sds((NQP, 8), f32)],
        grid=(NQP // KNN_BQ,),
        in_specs=[pl.BlockSpec((KNN_BQ, 3), lambda i: (i, 0)),
                  pl.BlockSpec((3, CP0), lambda i: (0, 0))],
        out_specs=[pl.BlockSpec((KNN_BQ, 8), lambda i: (i, 0)),
                   pl.BlockSpec((KNN_BQ, 8), lambda i: (i, 0))],
    )(pos1p, pos0t)

    idx_flat = jnp.pad(idx8[:N1, :3].reshape(-1), (0, GQ - 3 * N1))
    g = _knn_gather(idx_flat.reshape(-1, 128), x2)
    g3 = g[:3 * N1].reshape(N1, 3, H)

    y3, r3 = _call(
        _stage_interp_body, [_sds((N1, H)), _sds((N1, H))])(
        g3[:, 0], g3[:, 1], g3[:, 2], w8[:N1], pos_1,
        Wr2[:H], Wr2[H:], Wroot2[:H], Wroot2[H:], _row(br2))

    agg3 = _segsum1(arr1, y3, zeros1)[:, :N1]

    y4, r4 = _call(
        _stage_mid_body, [_sds((N1, 16)), _sds((N1, 16))])(
        agg3, r3, _row(bn_g3), _row(bn_b3), pos_1,
        wr3x, wr3p, wt3x, wt3p, br3p)

    agg4 = _segsum1s(arr1, y4, zeros1s)[:, :N1]

    out = _call(_stage_fin_body, _sds((N1, OUT)))(agg4, r4)
    return out
